# baseline (speedup 1.0000x reference)
import jax
import jax.numpy as jnp
from jax.experimental import pallas as pl
from jax.experimental.pallas import tpu as pltpu


def _merge3(a1, a2, a3, b1, b2, b3):
    s1 = jnp.minimum(a1, b1)
    x = jnp.maximum(a1, b1)
    y = jnp.minimum(a2, b2)
    s2 = jnp.minimum(x, y)
    s3 = jnp.minimum(jnp.maximum(x, y), jnp.minimum(a3, b3))
    return s1, s2, s3


def kernel(in_xyz, in_feature, out_xyz, out_feature, W1, b1, W2, b2):
    B, _, n_in = in_xyz.shape
    _, C, n_out = out_feature.shape
    T1 = 2048
    T2 = 4096
    nt1 = n_out // T1
    nt2 = n_out // T2
    P1 = B * nt1
    P2 = B * nt2
    inv_n = 1.0 / float(B * n_out)

    qpad = jnp.concatenate(
        [out_xyz, jnp.zeros((B, 4, n_out), jnp.float32),
         jnp.ones((B, 1, n_out), jnp.float32)], axis=1)
    ktpad = jnp.concatenate(
        [jnp.transpose(in_xyz, (0, 2, 1)) * (-2.0),
         jnp.zeros((B, n_in, 5), jnp.float32)], axis=2)

    def mega(q_ref, kt_ref, f_ref, of_ref, w1_ref, w2_ref, o_ref,
             ybuf, st1, st2):
        i = pl.program_id(0)

        @pl.when(i == 0)
        def _():
            st1[...] = jnp.zeros_like(st1)
            st2[...] = jnp.zeros_like(st2)

        @pl.when(i < P1)
        def _phase1():
            q = q_ref[0]
            kt = kt_ref[0]
            k2 = 0.25 * jnp.sum(kt * kt, axis=1, keepdims=True)
            e = jnp.dot(kt, q,
                        preferred_element_type=jnp.float32) + k2
            q2 = jnp.sum(q * q, axis=0, keepdims=True) - 1.0

            h = n_in // 2
            t1 = jnp.minimum(e[:h], e[h:])
            t2 = jnp.maximum(e[:h], e[h:])
            h //= 2
            a1, a2, b1_, b2_ = t1[:h], t2[:h], t1[h:], t2[h:]
            t1 = jnp.minimum(a1, b1_)
            x = jnp.maximum(a1, b1_)
            y = jnp.minimum(a2, b2_)
            t2 = jnp.minimum(x, y)
            t3 = jnp.maximum(x, y)
            while h > 8:
                h //= 2
                t1, t2, t3 = _merge3(t1[:h], t2[:h], t3[:h],
                                     t1[h:], t2[h:], t3[h:])
            for sh in (4, 2, 1):
                t1, t2, t3 = _merge3(t1, t2, t3,
                                     jnp.roll(t1, sh, axis=0),
                                     jnp.roll(t2, sh, axis=0),
                                     jnp.roll(t3, sh, axis=0))
            m1, m2, m3 = t1[0:1], t2[0:1], t3[0:1]

            i1 = 1.0 / jnp.maximum(m1 + q2, 1e-10)
            i2 = 1.0 / jnp.maximum(m2 + q2, 1e-10)
            i3 = 1.0 / jnp.maximum(m3 + q2, 1e-10)
            rtot = 1.0 / (i1 + i2 + i3)
            g = 1.0 / jnp.maximum(e + q2, 1e-10)
            s = jnp.where(e <= m3, g * rtot, 0.0).astype(jnp.bfloat16)

            interp = jnp.dot(f_ref[0], s, preferred_element_type=jnp.float32)
            xx = jnp.concatenate([interp.astype(jnp.bfloat16),
                                  of_ref[0]], axis=0)
            y1 = jnp.dot(w1_ref[...], xx, preferred_element_type=jnp.float32)
            ybuf[:, pl.ds(i * T1, T1)] = y1.astype(jnp.bfloat16)
            ssum = jnp.sum(y1, axis=1, keepdims=True)
            ssq = jnp.sum(y1 * y1, axis=1, keepdims=True)
            st1[...] += jnp.concatenate([ssum, ssq], axis=1)

        @pl.when((i >= P1) & (i < P1 + P2))
        def _phase2():
            j = i - P1
            yv = ybuf[:, pl.ds(j * T2, T2)].astype(jnp.float32)
            mean = st1[:, 0:1] * inv_n
            var = st1[:, 1:2] * inv_n - mean * mean
            rstd = jax.lax.rsqrt(var + 1e-5)
            x2 = jnp.maximum((yv - mean) * rstd, 0.0).astype(jnp.bfloat16)
            y2 = jnp.dot(w2_ref[...], x2, preferred_element_type=jnp.float32)
            ybuf[:, pl.ds(j * T2, T2)] = y2.astype(jnp.bfloat16)
            ssum = jnp.sum(y2, axis=1, keepdims=True)
            ssq = jnp.sum(y2 * y2, axis=1, keepdims=True)
            st2[...] += jnp.concatenate([ssum, ssq], axis=1)

        @pl.when(i >= P1 + P2)
        def _phase3():
            j = i - P1 - P2
            yv = ybuf[:, pl.ds(j * T2, T2)].astype(jnp.float32)
            mean = st2[:, 0:1] * inv_n
            var = st2[:, 1:2] * inv_n - mean * mean
            rstd = jax.lax.rsqrt(var + 1e-5)
            o_ref[0] = jnp.maximum((yv - mean) * rstd, 0.0)

    c1 = lambda i: (jnp.minimum(i, P1 - 1) // nt1, 0, jnp.minimum(i, P1 - 1) % nt1)
    cb = lambda i: (jnp.minimum(i, P1 - 1) // nt1, 0, 0)
    co = lambda i: (jnp.maximum(i - (P1 + P2), 0) // nt2, 0,
                    jnp.maximum(i - (P1 + P2), 0) % nt2)

    out = pl.pallas_call(
        mega,
        grid=(P1 + 2 * P2,),
        in_specs=[
            pl.BlockSpec((1, 8, T1), c1),
            pl.BlockSpec((1, n_in, 8), cb),
            pl.BlockSpec((1, C, n_in), cb),
            pl.BlockSpec((1, C, T1), c1),
            pl.BlockSpec((128, 128), lambda i: (0, 0)),
            pl.BlockSpec((128, 128), lambda i: (0, 0)),
        ],
        out_specs=pl.BlockSpec((1, 128, T2), co),
        out_shape=jax.ShapeDtypeStruct((B, 128, n_out), jnp.float32),
        scratch_shapes=[
            pltpu.VMEM((128, B * n_out), jnp.bfloat16),
            pltpu.VMEM((128, 2), jnp.float32),
            pltpu.VMEM((128, 2), jnp.float32),
        ],
    )(qpad, ktpad, in_feature.astype(jnp.bfloat16),
      out_feature.astype(jnp.bfloat16),
      W1.astype(jnp.bfloat16), W2.astype(jnp.bfloat16))
    return out


# speedup vs baseline: 1.0660x; 1.0660x over previous
import jax
import jax.numpy as jnp
from jax.experimental import pallas as pl
from jax.experimental.pallas import tpu as pltpu


def _merge3(a1, a2, a3, b1, b2, b3):
    s1 = jnp.minimum(a1, b1)
    x = jnp.maximum(a1, b1)
    y = jnp.minimum(a2, b2)
    s2 = jnp.minimum(x, y)
    s3 = jnp.minimum(jnp.maximum(x, y), jnp.minimum(a3, b3))
    return s1, s2, s3


def kernel(in_xyz, in_feature, out_xyz, out_feature, W1, b1, W2, b2):
    B, _, n_in = in_xyz.shape
    _, C, n_out = out_feature.shape
    T1 = 2048
    T2 = 4096
    nt1 = n_out // T1
    nt2 = n_out // T2
    P1 = B * nt1
    P2 = B * nt2
    inv_n = 1.0 / float(B * n_out)

    qpad = jnp.concatenate(
        [out_xyz, jnp.zeros((B, 4, n_out), jnp.float32),
         jnp.ones((B, 1, n_out), jnp.float32)], axis=1)
    ktpad = jnp.concatenate(
        [jnp.transpose(in_xyz, (0, 2, 1)) * (-2.0),
         jnp.zeros((B, n_in, 5), jnp.float32)], axis=2)

    def mega(q_ref, kt_ref, f_ref, of_ref, w1_ref, w2_ref, o_ref,
             ybuf, st1, st2):
        i = pl.program_id(0)

        @pl.when(i == 0)
        def _():
            st1[...] = jnp.zeros_like(st1)
            st2[...] = jnp.zeros_like(st2)

        @pl.when(i < P1)
        def _phase1():
            q = q_ref[0]
            kt = kt_ref[0]
            k2 = 0.25 * jnp.sum(kt * kt, axis=1, keepdims=True)
            e = jnp.dot(kt, q,
                        preferred_element_type=jnp.float32) + k2
            q2 = jnp.sum(q * q, axis=0, keepdims=True) - 1.0

            h = n_in // 2
            t1 = jnp.minimum(e[:h], e[h:])
            t2 = jnp.maximum(e[:h], e[h:])
            h //= 2
            a1, a2, b1_, b2_ = t1[:h], t2[:h], t1[h:], t2[h:]
            t1 = jnp.minimum(a1, b1_)
            x = jnp.maximum(a1, b1_)
            y = jnp.minimum(a2, b2_)
            t2 = jnp.minimum(x, y)
            t3 = jnp.maximum(x, y)
            while h > 8:
                h //= 2
                t1, t2, t3 = _merge3(t1[:h], t2[:h], t3[:h],
                                     t1[h:], t2[h:], t3[h:])
            for sh in (4, 2, 1):
                t1, t2, t3 = _merge3(t1, t2, t3,
                                     jnp.roll(t1, sh, axis=0),
                                     jnp.roll(t2, sh, axis=0),
                                     jnp.roll(t3, sh, axis=0))
            m1, m2, m3 = t1[0:1], t2[0:1], t3[0:1]

            i1 = 1.0 / jnp.maximum(m1 + q2, 1e-10)
            i2 = 1.0 / jnp.maximum(m2 + q2, 1e-10)
            i3 = 1.0 / jnp.maximum(m3 + q2, 1e-10)
            rtot = 1.0 / (i1 + i2 + i3)
            g = 1.0 / jnp.maximum(e + q2, 1e-10)
            s = jnp.where(e <= m3, g * rtot, 0.0).astype(jnp.bfloat16)

            interp = jnp.dot(f_ref[0], s, preferred_element_type=jnp.float32)
            xx = jnp.concatenate([interp, of_ref[0]], axis=0)
            y1 = jnp.dot(w1_ref[...], xx, preferred_element_type=jnp.float32)
            ybuf[:, pl.ds(i * T1, T1)] = y1.astype(jnp.bfloat16)
            ssum = jnp.sum(y1, axis=1, keepdims=True)
            ssq = jnp.sum(y1 * y1, axis=1, keepdims=True)
            st1[...] += jnp.concatenate([ssum, ssq], axis=1)

        @pl.when((i >= P1) & (i < P1 + P2))
        def _phase2():
            j = i - P1
            yv = ybuf[:, pl.ds(j * T2, T2)].astype(jnp.float32)
            mean = st1[:, 0:1] * inv_n
            var = st1[:, 1:2] * inv_n - mean * mean
            rstd = jax.lax.rsqrt(var + 1e-5)
            x2 = jnp.maximum((yv - mean) * rstd, 0.0)
            y2 = jnp.dot(w2_ref[...], x2, preferred_element_type=jnp.float32)
            ybuf[:, pl.ds(j * T2, T2)] = y2.astype(jnp.bfloat16)
            ssum = jnp.sum(y2, axis=1, keepdims=True)
            ssq = jnp.sum(y2 * y2, axis=1, keepdims=True)
            st2[...] += jnp.concatenate([ssum, ssq], axis=1)

        @pl.when(i >= P1 + P2)
        def _phase3():
            j = i - P1 - P2
            yv = ybuf[:, pl.ds(j * T2, T2)].astype(jnp.float32)
            mean = st2[:, 0:1] * inv_n
            var = st2[:, 1:2] * inv_n - mean * mean
            rstd = jax.lax.rsqrt(var + 1e-5)
            o_ref[0] = jnp.maximum((yv - mean) * rstd, 0.0)

    c1 = lambda i: (jnp.minimum(i, P1 - 1) // nt1, 0, jnp.minimum(i, P1 - 1) % nt1)
    cb = lambda i: (jnp.minimum(i, P1 - 1) // nt1, 0, 0)
    co = lambda i: (jnp.maximum(i - (P1 + P2), 0) // nt2, 0,
                    jnp.maximum(i - (P1 + P2), 0) % nt2)

    out = pl.pallas_call(
        mega,
        grid=(P1 + 2 * P2,),
        in_specs=[
            pl.BlockSpec((1, 8, T1), c1),
            pl.BlockSpec((1, n_in, 8), cb),
            pl.BlockSpec((1, C, n_in), cb),
            pl.BlockSpec((1, C, T1), c1),
            pl.BlockSpec((128, 128), lambda i: (0, 0)),
            pl.BlockSpec((128, 128), lambda i: (0, 0)),
        ],
        out_specs=pl.BlockSpec((1, 128, T2), co),
        out_shape=jax.ShapeDtypeStruct((B, 128, n_out), jnp.float32),
        scratch_shapes=[
            pltpu.VMEM((128, B * n_out), jnp.bfloat16),
            pltpu.VMEM((128, 2), jnp.float32),
            pltpu.VMEM((128, 2), jnp.float32),
        ],
    )(qpad, ktpad, in_feature.astype(jnp.bfloat16), out_feature, W1, W2)
    return out


# speedup vs baseline: 1.0873x; 1.0200x over previous
import jax
import jax.numpy as jnp
from jax.experimental import pallas as pl
from jax.experimental.pallas import tpu as pltpu


def _merge3(a1, a2, a3, b1, b2, b3):
    s1 = jnp.minimum(a1, b1)
    x = jnp.maximum(a1, b1)
    y = jnp.minimum(a2, b2)
    s2 = jnp.minimum(x, y)
    s3 = jnp.minimum(jnp.maximum(x, y), jnp.minimum(a3, b3))
    return s1, s2, s3


def kernel(in_xyz, in_feature, out_xyz, out_feature, W1, b1, W2, b2):
    B, _, n_in = in_xyz.shape
    _, C, n_out = out_feature.shape
    T1 = 2048
    T2 = 8192
    nt1 = n_out // T1
    nt2 = n_out // T2
    P1 = B * nt1
    P2 = B * nt2
    inv_n = 1.0 / float(B * n_out)

    qpad = jnp.concatenate(
        [out_xyz, jnp.zeros((B, 4, n_out), jnp.float32),
         jnp.ones((B, 1, n_out), jnp.float32)], axis=1)
    ktpad = jnp.concatenate(
        [jnp.transpose(in_xyz, (0, 2, 1)) * (-2.0),
         jnp.zeros((B, n_in, 5), jnp.float32)], axis=2)

    def mega(q_ref, kt_ref, f_ref, of_ref, w1_ref, w2_ref, o_ref,
             ybuf, st1, st2):
        i = pl.program_id(0)

        @pl.when(i == 0)
        def _():
            st1[...] = jnp.zeros_like(st1)
            st2[...] = jnp.zeros_like(st2)

        @pl.when(i < P1)
        def _phase1():
            q = q_ref[0]
            kt = kt_ref[0]
            k2 = 0.25 * jnp.sum(kt * kt, axis=1, keepdims=True)
            e = jnp.dot(kt, q,
                        preferred_element_type=jnp.float32) + k2
            q2 = jnp.sum(q * q, axis=0, keepdims=True) - 1.0

            h = n_in // 2
            t1 = jnp.minimum(e[:h], e[h:])
            t2 = jnp.maximum(e[:h], e[h:])
            h //= 2
            a1, a2, b1_, b2_ = t1[:h], t2[:h], t1[h:], t2[h:]
            t1 = jnp.minimum(a1, b1_)
            x = jnp.maximum(a1, b1_)
            y = jnp.minimum(a2, b2_)
            t2 = jnp.minimum(x, y)
            t3 = jnp.maximum(x, y)
            while h > 8:
                h //= 2
                t1, t2, t3 = _merge3(t1[:h], t2[:h], t3[:h],
                                     t1[h:], t2[h:], t3[h:])
            for sh in (4, 2, 1):
                t1, t2, t3 = _merge3(t1, t2, t3,
                                     jnp.roll(t1, sh, axis=0),
                                     jnp.roll(t2, sh, axis=0),
                                     jnp.roll(t3, sh, axis=0))
            m1, m2, m3 = t1[0:1], t2[0:1], t3[0:1]

            i1 = 1.0 / jnp.maximum(m1 + q2, 1e-10)
            i2 = 1.0 / jnp.maximum(m2 + q2, 1e-10)
            i3 = 1.0 / jnp.maximum(m3 + q2, 1e-10)
            rtot = 1.0 / (i1 + i2 + i3)
            g = 1.0 / jnp.maximum(e + q2, 1e-10)
            s = jnp.where(e <= m3, g * rtot, 0.0).astype(jnp.bfloat16)

            interp = jnp.dot(f_ref[0], s, preferred_element_type=jnp.float32)
            xx = jnp.concatenate([interp, of_ref[0]], axis=0)
            y1 = jnp.dot(w1_ref[...], xx, preferred_element_type=jnp.float32)
            ybuf[:, pl.ds(i * T1, T1)] = y1.astype(jnp.bfloat16)
            ssum = jnp.sum(y1, axis=1, keepdims=True)
            ssq = jnp.sum(y1 * y1, axis=1, keepdims=True)
            st1[...] += jnp.concatenate([ssum, ssq], axis=1)

        @pl.when((i >= P1) & (i < P1 + P2))
        def _phase2():
            j = i - P1
            yv = ybuf[:, pl.ds(j * T2, T2)].astype(jnp.float32)
            mean = st1[:, 0:1] * inv_n
            var = st1[:, 1:2] * inv_n - mean * mean
            rstd = jax.lax.rsqrt(var + 1e-5)
            x2 = jnp.maximum((yv - mean) * rstd, 0.0)
            y2 = jnp.dot(w2_ref[...], x2, preferred_element_type=jnp.float32)
            ybuf[:, pl.ds(j * T2, T2)] = y2.astype(jnp.bfloat16)
            ssum = jnp.sum(y2, axis=1, keepdims=True)
            ssq = jnp.sum(y2 * y2, axis=1, keepdims=True)
            st2[...] += jnp.concatenate([ssum, ssq], axis=1)

        @pl.when(i >= P1 + P2)
        def _phase3():
            j = i - P1 - P2
            yv = ybuf[:, pl.ds(j * T2, T2)].astype(jnp.float32)
            mean = st2[:, 0:1] * inv_n
            var = st2[:, 1:2] * inv_n - mean * mean
            rstd = jax.lax.rsqrt(var + 1e-5)
            o_ref[0] = jnp.maximum((yv - mean) * rstd, 0.0)

    c1 = lambda i: (jnp.minimum(i, P1 - 1) // nt1, 0, jnp.minimum(i, P1 - 1) % nt1)
    cb = lambda i: (jnp.minimum(i, P1 - 1) // nt1, 0, 0)
    co = lambda i: (jnp.maximum(i - (P1 + P2), 0) // nt2, 0,
                    jnp.maximum(i - (P1 + P2), 0) % nt2)

    out = pl.pallas_call(
        mega,
        grid=(P1 + 2 * P2,),
        in_specs=[
            pl.BlockSpec((1, 8, T1), c1),
            pl.BlockSpec((1, n_in, 8), cb),
            pl.BlockSpec((1, C, n_in), cb),
            pl.BlockSpec((1, C, T1), c1),
            pl.BlockSpec((128, 128), lambda i: (0, 0)),
            pl.BlockSpec((128, 128), lambda i: (0, 0)),
        ],
        out_specs=pl.BlockSpec((1, 128, T2), co),
        out_shape=jax.ShapeDtypeStruct((B, 128, n_out), jnp.float32),
        scratch_shapes=[
            pltpu.VMEM((128, B * n_out), jnp.bfloat16),
            pltpu.VMEM((128, 2), jnp.float32),
            pltpu.VMEM((128, 2), jnp.float32),
        ],
    )(qpad, ktpad, in_feature.astype(jnp.bfloat16), out_feature, W1, W2)
    return out
